# baseline (speedup 1.0000x reference)
import functools

import jax
import jax.numpy as jnp
from jax import lax
from jax.experimental import pallas as pl
from jax.experimental.pallas import tpu as pltpu
from jax.experimental.pallas import tpu_sc as plsc

_LAMB = 0.1
_CH = 16
_NCORE = 2
_NSUB = 16
_NW = _NCORE * _NSUB


_RING = 8
_CPP = 9


def _sc_pool(rt_ref, x_ref, buf, acc, sems, osem, *, B, T, D):
    w = lax.axis_index("s") * _NCORE + lax.axis_index("c")
    nd = D // 16
    ppw = (B * T // 2) // _NW
    G = ppw * _CPP

    def chunk_params(g):
        pair = g // _CPP
        cc = lax.rem(g, _CPP) if not isinstance(g, int) else g % _CPP
        p = w * ppw + pair
        b = p // (T // 2)
        q = lax.rem(p, T // 2)
        nA = q // _CH + 1
        isA = cc < nA
        k2 = jnp.where(isA, cc, cc - nA)
        r = jnp.where(isA, q, T - 1 - q)
        ln = jnp.where(isA, q + 1, T - q)
        slot = 2 * pair + jnp.where(isA, 0, 1)
        bound = (k2 + 1) * _CH - ln
        return b, r, k2, slot, bound

    def issue(g):
        b, r, k2, _, _ = chunk_params(g)
        par = lax.rem(g, _RING) if not isinstance(g, int) else g % _RING
        pltpu.make_async_copy(
            rt_ref.at[b, r, pl.ds(T - (k2 + 1) * _CH, _CH), :],
            buf.at[par],
            sems.at[par],
        ).start()

    z = jnp.zeros((16,), jnp.float32)
    for s in range(2 * ppw):
        for i in range(nd):
            acc[s, pl.ds(i * 16, 16)] = z

    for g0 in range(_RING):
        issue(g0)

    def group_body(grp, _):
        for par in range(_RING):
            g = grp * _RING + par
            _, _, _, slot, bound = chunk_params(g)
            pltpu.make_async_copy(
                rt_ref.at[0, 0, pl.ds(0, _CH), :], buf.at[par], sems.at[par]
            ).wait()
            regs = [z] * nd
            for jj in range(_CH):
                flag = (jj >= bound).astype(jnp.float32)
                for i in range(nd):
                    regs[i] = regs[i] + buf[par, jj, pl.ds(i * 16, 16)] * flag
            for i in range(nd):
                acc[slot, pl.ds(i * 16, 16)] = (
                    acc[slot, pl.ds(i * 16, 16)] + regs[i]
                )

            @pl.when(g + _RING < G)
            def _():
                issue(g + _RING)

        return 0

    lax.fori_loop(0, G // _RING, group_body, 0)

    def out_tasks():
        for k in range(ppw):
            p = w * ppw + k
            b = p // (T // 2)
            q = lax.rem(p, T // 2)
            yield 2 * k, b, q
            yield 2 * k + 1, b, T - 1 - q

    for slot, b, r in out_tasks():
        pltpu.make_async_copy(acc.at[slot], x_ref.at[b, r], osem).start()
    for slot, b, r in out_tasks():
        pltpu.make_async_copy(acc.at[slot], x_ref.at[b, r], osem).wait()


def _tc_attn(x_ref, centers_ref, val_ref, idx_ref, *, B, T, D, K):
    N = B * T
    x = x_ref[...].reshape(N, D)
    rows = lax.broadcasted_iota(jnp.int32, (N, 1), 0)
    seg = lax.rem(rows, T).astype(jnp.float32) + 1.0
    x = x / seg
    c_ = centers_ref[...]
    scale = 1.0 / jnp.sqrt(jnp.float32(D))
    logits = jax.lax.dot_general(
        x, c_, (((1,), (1,)), ((), ())), preferred_element_type=jnp.float32
    ) * scale
    m = jnp.max(logits, axis=1, keepdims=True)
    e = jnp.exp(logits - m)
    attn = e / jnp.sum(e, axis=1, keepdims=True)
    xq = jax.lax.dot_general(
        attn, c_, (((1,), (0,)), ((), ())), preferred_element_type=jnp.float32
    )
    xx = jnp.sum(xq * xq, axis=1, keepdims=True)
    cc = jnp.sum(c_ * c_, axis=1)
    xc = jax.lax.dot_general(
        xq, c_, (((1,), (1,)), ((), ())), preferred_element_type=jnp.float32
    )
    loss = xx - 2.0 * xc + cc[None, :] + _LAMB * (1.0 - seg)
    val = jnp.min(loss, axis=1)
    idx = jnp.argmin(loss, axis=1).astype(jnp.int32)
    for b in range(B):
        val_ref[b, :] = val[b * T:(b + 1) * T]
        idx_ref[b, :] = idx[b * T:(b + 1) * T]


def kernel(reps, rep_table, centers, timestep):
    B, T, D = reps.shape
    K = centers.shape[0]
    t = T
    start = timestep - t
    rt = jax.lax.dynamic_slice_in_dim(rep_table[:, :t], start, t, axis=2)

    mesh = plsc.VectorSubcoreMesh(core_axis_name="c", subcore_axis_name="s")
    x_sums = pl.kernel(
        functools.partial(_sc_pool, B=B, T=T, D=D),
        out_type=jax.ShapeDtypeStruct((B, T, D), jnp.float32),
        mesh=mesh,
        scratch_types=[
            pltpu.VMEM((_RING, _CH, D), jnp.float32),
            pltpu.VMEM((2 * (B * T // 2) // _NW, D), jnp.float32),
            pltpu.SemaphoreType.DMA((_RING,)),
            pltpu.SemaphoreType.DMA,
        ],
    )(rt)

    val, idx = pl.pallas_call(
        functools.partial(_tc_attn, B=B, T=T, D=D, K=K),
        in_specs=[
            pl.BlockSpec((B, T, D), lambda: (0, 0, 0)),
            pl.BlockSpec((K, D), lambda: (0, 0)),
        ],
        out_specs=[
            pl.BlockSpec((B, T), lambda: (0, 0)),
            pl.BlockSpec((B, T), lambda: (0, 0)),
        ],
        out_shape=[
            jax.ShapeDtypeStruct((B, T), jnp.float32),
            jax.ShapeDtypeStruct((B, T), jnp.int32),
        ],
    )(x_sums, centers)

    costs = jnp.full((B, T + 1), jnp.inf, jnp.float32)
    tokens = jnp.zeros((B, T + 1), jnp.int32)
    costs = jax.lax.dynamic_update_slice(costs, jnp.flip(val, axis=1), (0, start))
    tokens = jax.lax.dynamic_update_slice(tokens, jnp.flip(idx, axis=1), (0, start))
    return costs, tokens


# speedup vs baseline: 1.1886x; 1.1886x over previous
import functools

import jax
import jax.numpy as jnp
from jax import lax
from jax.experimental import pallas as pl
from jax.experimental.pallas import tpu as pltpu
from jax.experimental.pallas import tpu_sc as plsc

_LAMB = 0.1
_CH = 32
_NCORE = 2
_NSUB = 16
_NW = _NCORE * _NSUB


_RING = 4
_CPP = 5


def _sc_pool(rt_ref, x_ref, buf, acc, sems, osem, *, B, T, D):
    w = lax.axis_index("s") * _NCORE + lax.axis_index("c")
    nd = D // 16
    ppw = (B * T // 2) // _NW
    G = ppw * _CPP

    def chunk_params(g):
        pair = g // _CPP
        cc = lax.rem(g, _CPP) if not isinstance(g, int) else g % _CPP
        p = w * ppw + pair
        b = p // (T // 2)
        q = lax.rem(p, T // 2)
        nA = q // _CH + 1
        isA = cc < nA
        k2 = jnp.where(isA, cc, cc - nA)
        r = jnp.where(isA, q, T - 1 - q)
        ln = jnp.where(isA, q + 1, T - q)
        slot = 2 * pair + jnp.where(isA, 0, 1)
        bound = (k2 + 1) * _CH - ln
        return b, r, k2, slot, bound

    def issue(g):
        b, r, k2, _, _ = chunk_params(g)
        par = lax.rem(g, _RING) if not isinstance(g, int) else g % _RING
        pltpu.make_async_copy(
            rt_ref.at[b, r, pl.ds(T - (k2 + 1) * _CH, _CH), :],
            buf.at[par],
            sems.at[par],
        ).start()

    z = jnp.zeros((16,), jnp.float32)
    for s in range(2 * ppw):
        for i in range(nd):
            acc[s, pl.ds(i * 16, 16)] = z

    for g0 in range(_RING):
        issue(g0)

    def body(g, _):
        par = lax.rem(g, _RING)
        _, _, _, slot, bound = chunk_params(g)
        pltpu.make_async_copy(
            rt_ref.at[0, 0, pl.ds(0, _CH), :], buf.at[par], sems.at[par]
        ).wait()
        regs = [z] * nd
        for jj in range(_CH):
            flag = (jj >= bound).astype(jnp.float32)
            for i in range(nd):
                regs[i] = regs[i] + buf[par, jj, pl.ds(i * 16, 16)] * flag
        for i in range(nd):
            acc[slot, pl.ds(i * 16, 16)] = acc[slot, pl.ds(i * 16, 16)] + regs[i]

        @pl.when(g + _RING < G)
        def _():
            issue(g + _RING)

        return 0

    lax.fori_loop(0, G, body, 0)

    def out_tasks():
        for k in range(ppw):
            p = w * ppw + k
            b = p // (T // 2)
            q = lax.rem(p, T // 2)
            yield 2 * k, b, q
            yield 2 * k + 1, b, T - 1 - q

    for slot, b, r in out_tasks():
        pltpu.make_async_copy(acc.at[slot], x_ref.at[b, r], osem).start()
    for slot, b, r in out_tasks():
        pltpu.make_async_copy(acc.at[slot], x_ref.at[b, r], osem).wait()


def _tc_attn(x_ref, centers_ref, val_ref, idx_ref, *, B, T, D, K):
    N = B * T
    x = x_ref[...].reshape(N, D)
    rows = lax.broadcasted_iota(jnp.int32, (N, 1), 0)
    seg = lax.rem(rows, T).astype(jnp.float32) + 1.0
    x = x / seg
    c_ = centers_ref[...]
    scale = 1.0 / jnp.sqrt(jnp.float32(D))
    logits = jax.lax.dot_general(
        x, c_, (((1,), (1,)), ((), ())), preferred_element_type=jnp.float32
    ) * scale
    m = jnp.max(logits, axis=1, keepdims=True)
    e = jnp.exp(logits - m)
    attn = e / jnp.sum(e, axis=1, keepdims=True)
    xq = jax.lax.dot_general(
        attn, c_, (((1,), (0,)), ((), ())), preferred_element_type=jnp.float32
    )
    xx = jnp.sum(xq * xq, axis=1, keepdims=True)
    cc = jnp.sum(c_ * c_, axis=1)
    xc = jax.lax.dot_general(
        xq, c_, (((1,), (1,)), ((), ())), preferred_element_type=jnp.float32
    )
    loss = xx - 2.0 * xc + cc[None, :] + _LAMB * (1.0 - seg)
    val = jnp.min(loss, axis=1)
    idx = jnp.argmin(loss, axis=1).astype(jnp.int32)
    for b in range(B):
        val_ref[b, :] = val[b * T:(b + 1) * T]
        idx_ref[b, :] = idx[b * T:(b + 1) * T]


def kernel(reps, rep_table, centers, timestep):
    B, T, D = reps.shape
    K = centers.shape[0]
    t = T
    start = timestep - t
    rt = jax.lax.dynamic_slice_in_dim(rep_table[:, :t], start, t, axis=2)

    mesh = plsc.VectorSubcoreMesh(core_axis_name="c", subcore_axis_name="s")
    x_sums = pl.kernel(
        functools.partial(_sc_pool, B=B, T=T, D=D),
        out_type=jax.ShapeDtypeStruct((B, T, D), jnp.float32),
        mesh=mesh,
        scratch_types=[
            pltpu.VMEM((_RING, _CH, D), jnp.float32),
            pltpu.VMEM((2 * (B * T // 2) // _NW, D), jnp.float32),
            pltpu.SemaphoreType.DMA((_RING,)),
            pltpu.SemaphoreType.DMA,
        ],
    )(rt)

    val, idx = pl.pallas_call(
        functools.partial(_tc_attn, B=B, T=T, D=D, K=K),
        in_specs=[
            pl.BlockSpec((B, T, D), lambda: (0, 0, 0)),
            pl.BlockSpec((K, D), lambda: (0, 0)),
        ],
        out_specs=[
            pl.BlockSpec((B, T), lambda: (0, 0)),
            pl.BlockSpec((B, T), lambda: (0, 0)),
        ],
        out_shape=[
            jax.ShapeDtypeStruct((B, T), jnp.float32),
            jax.ShapeDtypeStruct((B, T), jnp.int32),
        ],
    )(x_sums, centers)

    costs = jnp.full((B, T + 1), jnp.inf, jnp.float32)
    tokens = jnp.zeros((B, T + 1), jnp.int32)
    costs = jax.lax.dynamic_update_slice(costs, jnp.flip(val, axis=1), (0, start))
    tokens = jax.lax.dynamic_update_slice(tokens, jnp.flip(idx, axis=1), (0, start))
    return costs, tokens


# speedup vs baseline: 4.2953x; 3.6139x over previous
import functools

import jax
import jax.numpy as jnp
from jax.experimental import pallas as pl
from jax.experimental.pallas import tpu as pltpu

_LAMB = 0.1
_RC = 8


def _chunk_copy(rt_hbm, bufs, sems, bb, c):
    T = rt_hbm.shape[1]
    j0 = T - _RC * (c + 1)
    return pltpu.make_async_copy(
        rt_hbm.at[bb, pl.ds(_RC * c, _RC), pl.ds(j0, _RC * (c + 1)), :],
        bufs[c],
        sems.at[c],
    )


def _kernel(rt_hbm, centers_ref, val_ref, idx_ref, *bufs_sems, T, K, D, B, NC):
    bufs = bufs_sems[:NC]
    x_ref = bufs_sems[NC]
    sems = bufs_sems[NC + 1]
    b = pl.program_id(0)

    @pl.when(b == 0)
    def _prologue():
        for c in range(NC):
            _chunk_copy(rt_hbm, bufs, sems, 0, c).start()

    rr = jax.lax.broadcasted_iota(jnp.int32, (_RC, _RC), 0)
    jj = jax.lax.broadcasted_iota(jnp.int32, (_RC, _RC), 1)
    keep = (jj >= _RC - 1 - rr).astype(jnp.float32)[:, :, None]

    for c in range(NC):
        _chunk_copy(rt_hbm, bufs, sems, b, c).wait()
        buf = bufs[c][...]
        x_rows = jnp.sum(buf[:, :_RC, :] * keep, axis=1)
        if c > 0:
            x_rows = x_rows + jnp.sum(buf[:, _RC:, :], axis=1)
        x_ref[pl.ds(_RC * c, _RC), :] = x_rows

        @pl.when(b + 1 < B)
        def _next():
            _chunk_copy(rt_hbm, bufs, sems, b + 1, c).start()

    rows = jax.lax.broadcasted_iota(jnp.int32, (T, 1), 0)
    seg = rows.astype(jnp.float32) + 1.0
    x = x_ref[...] / seg
    c_ = centers_ref[...]
    scale = 1.0 / jnp.sqrt(jnp.float32(D))
    logits = jax.lax.dot_general(
        x, c_, (((1,), (1,)), ((), ())), preferred_element_type=jnp.float32
    ) * scale
    m = jnp.max(logits, axis=1, keepdims=True)
    e = jnp.exp(logits - m)
    attn = e / jnp.sum(e, axis=1, keepdims=True)
    xq = jax.lax.dot_general(
        attn, c_, (((1,), (0,)), ((), ())), preferred_element_type=jnp.float32
    )
    xx = jnp.sum(xq * xq, axis=1, keepdims=True)
    cc = jnp.sum(c_ * c_, axis=1)
    xc = jax.lax.dot_general(
        xq, c_, (((1,), (1,)), ((), ())), preferred_element_type=jnp.float32
    )
    loss = xx - 2.0 * xc + cc[None, :] + _LAMB * (1.0 - seg)
    val_ref[b, :] = jnp.min(loss, axis=1)
    idx_ref[b, :] = jnp.argmin(loss, axis=1).astype(jnp.int32)


def kernel(reps, rep_table, centers, timestep):
    B, T, D = reps.shape
    K = centers.shape[0]
    t = T
    start = timestep - t
    rt = jax.lax.dynamic_slice_in_dim(rep_table[:, :t], start, t, axis=2)
    NC = T // _RC
    val, idx = pl.pallas_call(
        functools.partial(_kernel, T=T, K=K, D=D, B=B, NC=NC),
        grid=(B,),
        in_specs=[
            pl.BlockSpec(memory_space=pl.ANY),
            pl.BlockSpec((K, D), lambda b: (0, 0)),
        ],
        out_specs=[
            pl.BlockSpec((B, T), lambda b: (0, 0)),
            pl.BlockSpec((B, T), lambda b: (0, 0)),
        ],
        out_shape=[
            jax.ShapeDtypeStruct((B, T), jnp.float32),
            jax.ShapeDtypeStruct((B, T), jnp.int32),
        ],
        scratch_shapes=(
            [pltpu.VMEM((_RC, _RC * (c + 1), D), jnp.float32) for c in range(NC)]
            + [pltpu.VMEM((T, D), jnp.float32), pltpu.SemaphoreType.DMA((NC,))]
        ),
    )(rt, centers)
    costs = jnp.full((B, T + 1), jnp.inf, jnp.float32)
    tokens = jnp.zeros((B, T + 1), jnp.int32)
    costs = jax.lax.dynamic_update_slice(costs, jnp.flip(val, axis=1), (0, start))
    tokens = jax.lax.dynamic_update_slice(tokens, jnp.flip(idx, axis=1), (0, start))
    return costs, tokens
